# baseline (speedup 1.0000x reference)
import jax
import jax.numpy as jnp
from jax.experimental import pallas as pl
from jax.experimental.pallas import tpu as pltpu

BEAMS = 8
GROUPS = 32
VOCAB = 100000
CHUNK = 512
WIN = CHUNK + 128
TOTAL = 256 * VOCAB
NCHUNK = (VOCAB + CHUNK - 1) // CHUNK
NEG_INF = float("-inf")
BIG_I32 = 2**31 - 1


def _k1_maxsel(past_ref, cur_ref, awin_ref, col0_ref, base_ref, bias_ref):
    g = pl.program_id(0)
    x = cur_ref[...]
    maxes = []
    for c in range(NCHUNK):
        lo = c * CHUNK
        hi = min(lo + CHUNK, VOCAB)
        maxes.append(jnp.max(x[:, lo:hi], axis=1, keepdims=True))
    past = past_ref[...]
    vals = jnp.concatenate(maxes, axis=1) + past
    b = jax.lax.broadcasted_iota(jnp.int32, vals.shape, 0)
    c = jax.lax.broadcasted_iota(jnp.int32, vals.shape, 1)
    pid = b * NCHUNK + c
    b8 = jax.lax.broadcasted_iota(jnp.int32, past.shape, 0)

    awins = []
    col0s = []
    bases = []
    biases = []
    for _ in range(BEAMS):
        m = jnp.max(vals)
        sel = jnp.min(jnp.where(vals == m, pid, BIG_I32))
        bsel = sel // NCHUNK
        csel = sel % NCHUNK
        start = jnp.minimum(csel * CHUNK, VOCAB - CHUNK)
        row = g * BEAMS + bsel
        flat = row * VOCAB + start
        awin = jnp.minimum((flat // 128) * 128, TOTAL - WIN)
        awins.append(awin)
        col0s.append(awin - row * VOCAB)
        bases.append(bsel * VOCAB)
        biases.append(jnp.max(jnp.where(b8 == bsel, past, NEG_INF)))
        vals = jnp.where(pid == sel, NEG_INF, vals)
    awin_ref[...] = jnp.stack(awins).reshape(1, 1, BEAMS)
    col0_ref[...] = jnp.stack(col0s).reshape(1, BEAMS, 1)
    base_ref[...] = jnp.stack(bases).reshape(1, BEAMS, 1)
    bias_ref[...] = jnp.stack(biases).reshape(1, BEAMS, 1)


def _k2_gather_topk(awin_ref, cur_ref, col0_ref, base_ref, bias_ref,
                    topv_ref, sym_ref, vals_s, sem):
    copies = []
    for g in range(GROUPS):
        for j in range(BEAMS):
            i = g * BEAMS + j
            awin = pl.multiple_of(awin_ref[g, 0, j], 128)
            cp = pltpu.make_async_copy(
                cur_ref.at[:, pl.ds(awin, WIN)],
                vals_s.at[pl.ds(i, 1), :],
                sem,
            )
            cp.start()
            copies.append(cp)
    for cp in copies:
        cp.wait()

    raw = vals_s[...].reshape(GROUPS, BEAMS, WIN)
    lane = jax.lax.broadcasted_iota(jnp.int32, raw.shape, 2)
    col = col0_ref[...] + lane
    valid = (col >= 0) & (col < VOCAB)
    vals = jnp.where(valid, raw + bias_ref[...], NEG_INF)
    gidx = jnp.where(valid, base_ref[...] + col, BIG_I32)
    tv = []
    ts = []
    for _ in range(BEAMS):
        m = jnp.max(vals, axis=(1, 2), keepdims=True)
        sel = jnp.min(jnp.where(vals == m, gidx, BIG_I32),
                      axis=(1, 2), keepdims=True)
        tv.append(m)
        ts.append(sel)
        vals = jnp.where(gidx == sel, NEG_INF, vals)
    topv_ref[...] = jnp.concatenate(tv, axis=2)
    sym_ref[...] = jnp.concatenate(ts, axis=2) % VOCAB


@jax.jit
def _run(past_p, cur_p):
    awin, col0, base, bias = pl.pallas_call(
        _k1_maxsel,
        grid=(GROUPS,),
        in_specs=[
            pl.BlockSpec((BEAMS, 1), lambda g: (g, 0)),
            pl.BlockSpec((BEAMS, VOCAB), lambda g: (g, 0)),
        ],
        out_specs=[
            pl.BlockSpec((1, 1, BEAMS), lambda g: (g, 0, 0)),
            pl.BlockSpec((1, BEAMS, 1), lambda g: (g, 0, 0)),
            pl.BlockSpec((1, BEAMS, 1), lambda g: (g, 0, 0)),
            pl.BlockSpec((1, BEAMS, 1), lambda g: (g, 0, 0)),
        ],
        out_shape=(
            jax.ShapeDtypeStruct((GROUPS, 1, BEAMS), jnp.int32),
            jax.ShapeDtypeStruct((GROUPS, BEAMS, 1), jnp.int32),
            jax.ShapeDtypeStruct((GROUPS, BEAMS, 1), jnp.int32),
            jax.ShapeDtypeStruct((GROUPS, BEAMS, 1), jnp.float32),
        ),
        compiler_params=pltpu.CompilerParams(
            dimension_semantics=("arbitrary",),
        ),
    )(past_p, cur_p)

    grid_spec = pltpu.PrefetchScalarGridSpec(
        num_scalar_prefetch=1,
        grid=(1,),
        in_specs=[
            pl.BlockSpec(memory_space=pl.MemorySpace.ANY),
            pl.BlockSpec((GROUPS, BEAMS, 1), lambda i, *_: (0, 0, 0)),
            pl.BlockSpec((GROUPS, BEAMS, 1), lambda i, *_: (0, 0, 0)),
            pl.BlockSpec((GROUPS, BEAMS, 1), lambda i, *_: (0, 0, 0)),
        ],
        out_specs=[
            pl.BlockSpec((GROUPS, 1, BEAMS), lambda i, *_: (0, 0, 0)),
            pl.BlockSpec((GROUPS, 1, BEAMS), lambda i, *_: (0, 0, 0)),
        ],
        scratch_shapes=[
            pltpu.VMEM((GROUPS * BEAMS, WIN), jnp.float32),
            pltpu.SemaphoreType.DMA,
        ],
    )

    topv, sym = pl.pallas_call(
        _k2_gather_topk,
        grid_spec=grid_spec,
        out_shape=(
            jax.ShapeDtypeStruct((GROUPS, 1, BEAMS), jnp.float32),
            jax.ShapeDtypeStruct((GROUPS, 1, BEAMS), jnp.int32),
        ),
    )(awin, cur_p.reshape(1, -1), col0, base, bias)

    return topv.reshape(-1, 1), sym.reshape(GROUPS, BEAMS)


def kernel(past_p, cur_p, batch_size, step):
    del batch_size, step
    return _run(past_p, cur_p)


# speedup vs baseline: 1.5377x; 1.5377x over previous
import jax
import jax.numpy as jnp
from jax.experimental import pallas as pl
from jax.experimental.pallas import tpu as pltpu

BEAMS = 8
GROUPS = 32
VOCAB = 100000
CHUNK = 512
NCHUNK = (VOCAB + CHUNK - 1) // CHUNK
NEG_INF = float("-inf")
BIG_I32 = 2**31 - 1


def _k1_maxsel(past_ref, cur_ref, rows_ref, cols_ref, start_ref, base_ref,
               bias_ref):
    g = pl.program_id(0)
    x = cur_ref[...]
    maxes = []
    for c in range(NCHUNK):
        lo = c * CHUNK
        hi = min(lo + CHUNK, VOCAB)
        maxes.append(jnp.max(x[:, lo:hi], axis=1, keepdims=True))
    past = past_ref[...]
    vals = jnp.concatenate(maxes, axis=1) + past
    b = jax.lax.broadcasted_iota(jnp.int32, vals.shape, 0)
    c = jax.lax.broadcasted_iota(jnp.int32, vals.shape, 1)
    pid = b * NCHUNK + c
    b8 = jax.lax.broadcasted_iota(jnp.int32, past.shape, 0)

    rows = []
    cols = []
    starts = []
    bases = []
    biases = []
    for _ in range(BEAMS):
        m = jnp.max(vals)
        sel = jnp.min(jnp.where(vals == m, pid, BIG_I32))
        bsel = sel // NCHUNK
        csel = sel % NCHUNK
        rows.append(g * BEAMS + bsel)
        cols.append(csel)
        starts.append(csel * CHUNK)
        bases.append(bsel * VOCAB + csel * CHUNK)
        biases.append(jnp.max(jnp.where(b8 == bsel, past, NEG_INF)))
        vals = jnp.where(pid == sel, NEG_INF, vals)
    rows_ref[...] = jnp.stack(rows).reshape(1, 1, BEAMS)
    cols_ref[...] = jnp.stack(cols).reshape(1, 1, BEAMS)
    start_ref[...] = jnp.stack(starts).reshape(1, BEAMS, 1)
    base_ref[...] = jnp.stack(bases).reshape(1, BEAMS, 1)
    bias_ref[...] = jnp.stack(biases).reshape(1, BEAMS, 1)


def _k2_gather_topk(rows_ref, cols_ref, *refs):
    del rows_ref, cols_ref
    cur_refs = refs[:BEAMS]
    start_ref, base_ref, bias_ref = refs[BEAMS:BEAMS + 3]
    topv_ref, sym_ref = refs[BEAMS + 3], refs[BEAMS + 4]

    raw = jnp.concatenate(
        [cur_refs[j][...].reshape(1, CHUNK) for j in range(BEAMS)], axis=0)
    lane = jax.lax.broadcasted_iota(jnp.int32, raw.shape, 1)
    start = start_ref[...].reshape(BEAMS, 1)
    base = base_ref[...].reshape(BEAMS, 1)
    bias = bias_ref[...].reshape(BEAMS, 1)
    valid = (start + lane) < VOCAB
    vals = jnp.where(valid, raw + bias, NEG_INF)
    gidx = jnp.where(valid, base + lane, BIG_I32)

    tv = []
    ts = []
    for _ in range(BEAMS):
        m = jnp.max(vals)
        sel = jnp.min(jnp.where(vals == m, gidx, BIG_I32))
        tv.append(m)
        ts.append(sel)
        vals = jnp.where(gidx == sel, NEG_INF, vals)
    topv_ref[...] = jnp.stack(tv).reshape(1, 1, BEAMS)
    sym_ref[...] = (jnp.stack(ts) % VOCAB).reshape(1, 1, BEAMS)


@jax.jit
def _run(past_p, cur_p):
    rows, cols, start, base, bias = pl.pallas_call(
        _k1_maxsel,
        grid=(GROUPS,),
        in_specs=[
            pl.BlockSpec((BEAMS, 1), lambda g: (g, 0)),
            pl.BlockSpec((BEAMS, VOCAB), lambda g: (g, 0)),
        ],
        out_specs=[
            pl.BlockSpec((1, 1, BEAMS), lambda g: (g, 0, 0)),
            pl.BlockSpec((1, 1, BEAMS), lambda g: (g, 0, 0)),
            pl.BlockSpec((1, BEAMS, 1), lambda g: (g, 0, 0)),
            pl.BlockSpec((1, BEAMS, 1), lambda g: (g, 0, 0)),
            pl.BlockSpec((1, BEAMS, 1), lambda g: (g, 0, 0)),
        ],
        out_shape=(
            jax.ShapeDtypeStruct((GROUPS, 1, BEAMS), jnp.int32),
            jax.ShapeDtypeStruct((GROUPS, 1, BEAMS), jnp.int32),
            jax.ShapeDtypeStruct((GROUPS, BEAMS, 1), jnp.int32),
            jax.ShapeDtypeStruct((GROUPS, BEAMS, 1), jnp.int32),
            jax.ShapeDtypeStruct((GROUPS, BEAMS, 1), jnp.float32),
        ),
        compiler_params=pltpu.CompilerParams(
            dimension_semantics=("parallel",),
        ),
    )(past_p, cur_p)

    grid_spec = pltpu.PrefetchScalarGridSpec(
        num_scalar_prefetch=2,
        grid=(GROUPS,),
        in_specs=(
            [
                pl.BlockSpec(
                    (1, 1, CHUNK),
                    (lambda g, rows_ref, cols_ref, j=j:
                     (rows_ref[g, 0, j], 0, cols_ref[g, 0, j])),
                )
                for j in range(BEAMS)
            ]
            + [pl.BlockSpec((1, BEAMS, 1), lambda g, *_: (g, 0, 0))] * 3
        ),
        out_specs=[
            pl.BlockSpec((1, 1, BEAMS), lambda g, *_: (g, 0, 0)),
            pl.BlockSpec((1, 1, BEAMS), lambda g, *_: (g, 0, 0)),
        ],
    )

    topv, sym = pl.pallas_call(
        _k2_gather_topk,
        grid_spec=grid_spec,
        out_shape=(
            jax.ShapeDtypeStruct((GROUPS, 1, BEAMS), jnp.float32),
            jax.ShapeDtypeStruct((GROUPS, 1, BEAMS), jnp.int32),
        ),
        compiler_params=pltpu.CompilerParams(
            dimension_semantics=("arbitrary",),
        ),
    )(rows, cols, *([cur_p.reshape(256, 1, VOCAB)] * BEAMS), start, base, bias)

    return topv.reshape(-1, 1), sym.reshape(GROUPS, BEAMS)


def kernel(past_p, cur_p, batch_size, step):
    del batch_size, step
    return _run(past_p, cur_p)


# speedup vs baseline: 2.8291x; 1.8399x over previous
import jax
import jax.numpy as jnp
from jax.experimental import pallas as pl
from jax.experimental.pallas import tpu as pltpu

BEAMS = 8
GROUPS = 32
VOCAB = 100000
CHUNK = 512
WIN = 544
NCHUNK = (VOCAB + CHUNK - 1) // CHUNK
NEG_INF = float("-inf")
BIG_I32 = 2**31 - 1


def _fused(past_ref, cur_ref, topv_ref, sym_ref):
    maxes = []
    for c in range(NCHUNK):
        lo = c * CHUNK
        hi = min(lo + CHUNK, VOCAB)
        maxes.append(jnp.max(cur_ref[:, lo:hi], axis=1, keepdims=True))
    past = past_ref[...]
    cmax = jnp.concatenate(maxes, axis=1) + past
    b = jax.lax.broadcasted_iota(jnp.int32, cmax.shape, 0)
    c = jax.lax.broadcasted_iota(jnp.int32, cmax.shape, 1)
    pid = b * NCHUNK + c

    win_vals = []
    win_gidx = []
    lane = jax.lax.broadcasted_iota(jnp.int32, (BEAMS, WIN), 1)
    b8 = jax.lax.broadcasted_iota(jnp.int32, (BEAMS, WIN), 0)
    for _ in range(BEAMS):
        m = jnp.max(cmax)
        sel = jnp.min(jnp.where(cmax == m, pid, BIG_I32))
        csel = sel % NCHUNK
        start = pl.multiple_of(jnp.minimum(csel * CHUNK, VOCAB - WIN), 128)
        w = cur_ref[:, pl.ds(start, WIN)] + past
        win_vals.append(w)
        win_gidx.append(b8 * VOCAB + start + lane)
        cmax = jnp.where(pid == sel, NEG_INF, cmax)

    vals = jnp.concatenate(win_vals, axis=1)
    gidx = jnp.concatenate(win_gidx, axis=1)
    tv = []
    ts = []
    for _ in range(BEAMS):
        m = jnp.max(vals)
        sel = jnp.min(jnp.where(vals == m, gidx, BIG_I32))
        tv.append(m)
        ts.append(sel)
        vals = jnp.where(gidx == sel, NEG_INF, vals)
    topv_ref[...] = jnp.stack(tv).reshape(1, 1, BEAMS)
    sym_ref[...] = (jnp.stack(ts) % VOCAB).reshape(1, 1, BEAMS)


@jax.jit
def _run(past_p, cur_p):
    topv, sym = pl.pallas_call(
        _fused,
        grid=(GROUPS,),
        in_specs=[
            pl.BlockSpec((BEAMS, 1), lambda g: (g, 0)),
            pl.BlockSpec((BEAMS, VOCAB), lambda g: (g, 0)),
        ],
        out_specs=[
            pl.BlockSpec((1, 1, BEAMS), lambda g: (g, 0, 0)),
            pl.BlockSpec((1, 1, BEAMS), lambda g: (g, 0, 0)),
        ],
        out_shape=(
            jax.ShapeDtypeStruct((GROUPS, 1, BEAMS), jnp.float32),
            jax.ShapeDtypeStruct((GROUPS, 1, BEAMS), jnp.int32),
        ),
        compiler_params=pltpu.CompilerParams(
            dimension_semantics=("parallel",),
        ),
    )(past_p, cur_p)

    return topv.reshape(-1, 1), sym.reshape(GROUPS, BEAMS)


def kernel(past_p, cur_p, batch_size, step):
    del batch_size, step
    return _run(past_p, cur_p)


# speedup vs baseline: 2.9428x; 1.0402x over previous
import jax
import jax.numpy as jnp
from jax.experimental import pallas as pl
from jax.experimental.pallas import tpu as pltpu

BEAMS = 8
GROUPS = 32
GPB = 4
ROWS = BEAMS * GPB
VOCAB = 100000
CHUNK = 512
WIN = 544
NCHUNK = (VOCAB + CHUNK - 1) // CHUNK
NEG_INF = float("-inf")
BIG_I32 = 2**31 - 1


def _fused(past_ref, cur_ref, topv_ref, sym_ref):
    maxes = []
    for c in range(NCHUNK):
        lo = c * CHUNK
        hi = min(lo + CHUNK, VOCAB)
        maxes.append(jnp.max(cur_ref[:, lo:hi], axis=1, keepdims=True))
    past = past_ref[...]
    cmax_all = jnp.concatenate(maxes, axis=1) + past

    pid = (jax.lax.broadcasted_iota(jnp.int32, (BEAMS, NCHUNK), 0) * NCHUNK
           + jax.lax.broadcasted_iota(jnp.int32, (BEAMS, NCHUNK), 1))
    b8 = jax.lax.broadcasted_iota(jnp.int32, (BEAMS, 1), 0)
    bw = jax.lax.broadcasted_iota(jnp.int32, (BEAMS, WIN), 0)
    lane = jax.lax.broadcasted_iota(jnp.int32, (1, WIN), 1)

    tv_groups = []
    ts_groups = []
    for gg in range(GPB):
        r0 = gg * BEAMS
        cmax = cmax_all[r0:r0 + BEAMS, :]
        pgroup = past[r0:r0 + BEAMS, :]

        pool_vals = []
        pool_gidx = []
        for _ in range(BEAMS):
            m = jnp.max(cmax)
            sel = jnp.min(jnp.where(cmax == m, pid, BIG_I32))
            bsel = sel // NCHUNK
            csel = sel % NCHUNK
            start = pl.multiple_of(
                jnp.minimum(csel * CHUNK, VOCAB - WIN), 128)
            w = cur_ref[r0:r0 + BEAMS, pl.ds(start, WIN)]
            bias = jnp.max(jnp.where(b8 == bsel, pgroup, NEG_INF))
            row = jnp.max(jnp.where(bw == bsel, w, NEG_INF),
                          axis=0, keepdims=True) + bias
            pool_vals.append(row)
            pool_gidx.append(bsel * VOCAB + start + lane)
            cmax = jnp.where(pid == sel, NEG_INF, cmax)

        vals = jnp.concatenate(pool_vals, axis=0)
        gidx = jnp.concatenate(pool_gidx, axis=0)
        tv = []
        ts = []
        for _ in range(BEAMS):
            m = jnp.max(vals)
            sel = jnp.min(jnp.where(vals == m, gidx, BIG_I32))
            tv.append(m)
            ts.append(sel)
            vals = jnp.where(gidx == sel, NEG_INF, vals)
        tv_groups.append(jnp.stack(tv).reshape(1, 1, BEAMS))
        ts_groups.append((jnp.stack(ts) % VOCAB).reshape(1, 1, BEAMS))

    topv_ref[...] = jnp.concatenate(tv_groups, axis=0)
    sym_ref[...] = jnp.concatenate(ts_groups, axis=0)


@jax.jit
def _run(past_p, cur_p):
    topv, sym = pl.pallas_call(
        _fused,
        grid=(GROUPS // GPB,),
        in_specs=[
            pl.BlockSpec((ROWS, 1), lambda g: (g, 0)),
            pl.BlockSpec((ROWS, VOCAB), lambda g: (g, 0)),
        ],
        out_specs=[
            pl.BlockSpec((GPB, 1, BEAMS), lambda g: (g, 0, 0)),
            pl.BlockSpec((GPB, 1, BEAMS), lambda g: (g, 0, 0)),
        ],
        out_shape=(
            jax.ShapeDtypeStruct((GROUPS, 1, BEAMS), jnp.float32),
            jax.ShapeDtypeStruct((GROUPS, 1, BEAMS), jnp.int32),
        ),
        compiler_params=pltpu.CompilerParams(
            dimension_semantics=("parallel",),
        ),
    )(past_p, cur_p)

    return topv.reshape(-1, 1), sym.reshape(GROUPS, BEAMS)


def kernel(past_p, cur_p, batch_size, step):
    del batch_size, step
    return _run(past_p, cur_p)


# speedup vs baseline: 3.8870x; 1.3209x over previous
import jax
import jax.numpy as jnp
from jax.experimental import pallas as pl
from jax.experimental.pallas import tpu as pltpu

BEAMS = 8
GROUPS = 32
GPB = 4
ROWS = BEAMS * GPB
VOCAB = 100000
CHUNK = 512
WIN = 544
NCHUNK = (VOCAB + CHUNK - 1) // CHUNK
NEG_INF = float("-inf")
BIG_I32 = 2**31 - 1


def _fused(past_ref, cur_ref, topv_ref, sym_ref):
    maxes = []
    for c in range(NCHUNK):
        lo = c * CHUNK
        hi = min(lo + CHUNK, VOCAB)
        maxes.append(jnp.max(cur_ref[:, lo:hi], axis=1, keepdims=True))
    past = past_ref[...]
    cmax_all = jnp.concatenate(maxes, axis=1) + past

    pid = (jax.lax.broadcasted_iota(jnp.int32, (BEAMS, NCHUNK), 0) * NCHUNK
           + jax.lax.broadcasted_iota(jnp.int32, (BEAMS, NCHUNK), 1))
    b8 = jax.lax.broadcasted_iota(jnp.int32, (BEAMS, 1), 0)
    bw = jax.lax.broadcasted_iota(jnp.int32, (BEAMS, WIN), 0)
    lane = jax.lax.broadcasted_iota(jnp.int32, (1, WIN), 1)

    pool_vals = []
    pool_gidx = []
    for gg in range(GPB):
        r0 = gg * BEAMS
        cmax = cmax_all[r0:r0 + BEAMS, :]
        pgroup = past[r0:r0 + BEAMS, :]
        for _ in range(BEAMS):
            m = jnp.max(cmax)
            sel = jnp.min(jnp.where(cmax == m, pid, BIG_I32))
            bsel = sel // NCHUNK
            csel = sel % NCHUNK
            start = pl.multiple_of(
                jnp.minimum(csel * CHUNK, VOCAB - WIN), 128)
            w = cur_ref[r0:r0 + BEAMS, pl.ds(start, WIN)]
            bias = jnp.max(jnp.where(b8 == bsel, pgroup, NEG_INF))
            row = jnp.max(jnp.where(bw == bsel, w, NEG_INF),
                          axis=0, keepdims=True) + bias
            pool_vals.append(row)
            pool_gidx.append(bsel * VOCAB + start + lane)
            cmax = jnp.where(pid == sel, NEG_INF, cmax)

    vals = jnp.concatenate(pool_vals, axis=0).reshape(GPB, BEAMS, WIN)
    gidx = jnp.concatenate(pool_gidx, axis=0).reshape(GPB, BEAMS, WIN)
    tv = []
    ts = []
    for _ in range(BEAMS):
        m = jnp.max(vals, axis=(1, 2), keepdims=True)
        sel = jnp.min(jnp.where(vals == m, gidx, BIG_I32),
                      axis=(1, 2), keepdims=True)
        tv.append(m)
        ts.append(sel)
        vals = jnp.where(gidx == sel, NEG_INF, vals)
    topv_ref[...] = jnp.concatenate(tv, axis=2)
    sym_ref[...] = jnp.concatenate(ts, axis=2) % VOCAB


@jax.jit
def _run(past_p, cur_p):
    topv, sym = pl.pallas_call(
        _fused,
        grid=(GROUPS // GPB,),
        in_specs=[
            pl.BlockSpec((ROWS, 1), lambda g: (g, 0)),
            pl.BlockSpec((ROWS, VOCAB), lambda g: (g, 0)),
        ],
        out_specs=[
            pl.BlockSpec((GPB, 1, BEAMS), lambda g: (g, 0, 0)),
            pl.BlockSpec((GPB, 1, BEAMS), lambda g: (g, 0, 0)),
        ],
        out_shape=(
            jax.ShapeDtypeStruct((GROUPS, 1, BEAMS), jnp.float32),
            jax.ShapeDtypeStruct((GROUPS, 1, BEAMS), jnp.int32),
        ),
        compiler_params=pltpu.CompilerParams(
            dimension_semantics=("parallel",),
        ),
    )(past_p, cur_p)

    return topv.reshape(-1, 1), sym.reshape(GROUPS, BEAMS)


def kernel(past_p, cur_p, batch_size, step):
    del batch_size, step
    return _run(past_p, cur_p)
